# Initial kernel scaffold; baseline (speedup 1.0000x reference)
import jax
import jax.numpy as jnp
from jax.experimental import pallas as pl


def kernel(x_cell, edge_index_spatial, edge_index_similar, proj_w1, proj_b1, att_src_sp1, att_dst_sp1, att_src_sim1, att_dst_sim1, k_w1, k_b1, q1, proj_w2, proj_b2, att_src_sp2, att_dst_sp2, att_src_sim2, att_dst_sim2, k_w2, k_b2, q2, lin_w, lin_b):
    raise NotImplementedError("write your pallas kernel here")


# speedup vs baseline: 1.0429x; 1.0429x over previous
import functools

import jax
import jax.numpy as jnp
from jax.experimental import pallas as pl
from jax.experimental.pallas import tpu as pltpu

N = 50000
E = 400000
HID = 128
HEADS = 2
DH = HID // HEADS
OUT = 16

NB = 512


def _matmul_block(x_ref, w_ref, b_ref, o_ref):
    o_ref[...] = jnp.dot(x_ref[...], w_ref[...],
                         preferred_element_type=jnp.float32) + b_ref[...]


def _proj(x, w, b):
    n = x.shape[0]
    pad = (-n) % NB
    xp = jnp.pad(x, ((0, pad), (0, 0)))
    grid = (xp.shape[0] // NB,)
    out = pl.pallas_call(
        _matmul_block,
        grid=grid,
        in_specs=[pl.BlockSpec((NB, x.shape[1]), lambda i: (i, 0)),
                  pl.BlockSpec((x.shape[1], w.shape[1]), lambda i: (0, 0)),
                  pl.BlockSpec((w.shape[1],), lambda i: (0,))],
        out_specs=pl.BlockSpec((NB, w.shape[1]), lambda i: (i, 0)),
        out_shape=jax.ShapeDtypeStruct((xp.shape[0], w.shape[1]), jnp.float32),
    )(xp, w, b)
    return out[:n]


def _han_conv(x, edges, atts, proj_w, proj_b, k_w, k_b, q):
    n = x.shape[0]
    h = _proj(x, proj_w, proj_b).reshape(n, HEADS, DH)
    outs = []
    for ei, (a_s, a_d) in zip(edges, atts):
        src = ei[0]
        dst = ei[1]
        alpha_src = (h * a_s).sum(-1)
        alpha_dst = (h * a_d).sum(-1)
        alpha = alpha_src[src] + alpha_dst[dst]
        alpha = jax.nn.leaky_relu(alpha, 0.2)
        ex = jnp.exp(alpha)
        denom = jax.ops.segment_sum(ex, dst, num_segments=n)
        msg = h[src] * ex[..., None]
        out = jax.ops.segment_sum(msg, dst, num_segments=n).reshape(n, HID)
        out = out / jnp.clip(denom, 1e-16).repeat(DH, axis=1)
        outs.append(jax.nn.relu(out))
    stacked = jnp.stack(outs)
    w = jnp.tanh(stacked @ k_w + k_b).mean(axis=1)
    score = (w * q).sum(-1)
    attn = jax.nn.softmax(score)
    return (attn[:, None, None] * stacked).sum(0)


def kernel(x_cell, edge_index_spatial, edge_index_similar, proj_w1, proj_b1,
           att_src_sp1, att_dst_sp1, att_src_sim1, att_dst_sim1, k_w1, k_b1,
           q1, proj_w2, proj_b2, att_src_sp2, att_dst_sp2, att_src_sim2,
           att_dst_sim2, k_w2, k_b2, q2, lin_w, lin_b):
    edges = [edge_index_spatial, edge_index_similar]
    h = _han_conv(x_cell, edges,
                  [(att_src_sp1, att_dst_sp1), (att_src_sim1, att_dst_sim1)],
                  proj_w1, proj_b1, k_w1, k_b1, q1)
    h = jax.nn.relu(h)
    h = _han_conv(h, edges,
                  [(att_src_sp2, att_dst_sp2), (att_src_sim2, att_dst_sim2)],
                  proj_w2, proj_b2, k_w2, k_b2, q2)
    h = jax.nn.relu(h)
    return _proj(h, lin_w, lin_b)


# speedup vs baseline: 22.2017x; 21.2885x over previous
import jax
import jax.numpy as jnp
from jax import lax
from jax.experimental import pallas as pl
from jax.experimental.pallas import tpu as pltpu
from jax.experimental.pallas import tpu_sc as plsc

NN = 50000
EE = 400000
HID = 128
DH = 64
OUTD = 16

NC, NS, L = 2, 16, 16
NCK = 8
CH = 6272
NP = NCK * CH
TRASH = CH
TPR = 400
ACC_R = NS * TPR
CAP = 12800
NBLK = CAP // 128
ET = 12512
LDM = 0x1FFF
NB = 512
GB = NP // NB

_mesh = plsc.VectorSubcoreMesh(core_axis_name="c", subcore_axis_name="s")
_params = pltpu.CompilerParams(needs_layout_passes=False)


def _bin_body(src_hbm, dst_hbm, pk, cnts, src_v, dst_v, ob, cv):
    wid = lax.axis_index("s") * NC + lax.axis_index("c")
    pltpu.sync_copy(src_hbm.at[wid], src_v)
    pltpu.sync_copy(dst_hbm.at[wid], dst_v)
    io = lax.iota(jnp.int32, L)
    cvec = jnp.zeros((L,), jnp.int32)
    for b in range(NCK):
        def body(gidx, off):
            d = dst_v[pl.ds(gidx * L, L)]
            s = src_v[pl.ds(gidx * L, L)]
            ch = d // CH
            pkw = jnp.left_shift(s, 16) | (d - ch * CH)
            m = ch == b
            plsc.store_compressed(ob.at[pl.ds(off, L)], pkw, mask=m)
            return off + jnp.sum(m.astype(jnp.int32))
        cnt = lax.fori_loop(0, ET // L, body, 0)
        trash = jnp.full((L,), TRASH, jnp.int32)
        for k in range(17):
            ob[pl.ds(cnt + L * k, L)] = trash
        padded = ((cnt + 255) // 256) * 256
        cvec = jnp.where(io == b, padded, cvec)
        pltpu.sync_copy(ob, pk.at[b, wid])
    cv[...] = cvec
    pltpu.sync_copy(cv, cnts.at[wid])


_bin_kernel = pl.kernel(
    _bin_body,
    out_type=[jax.ShapeDtypeStruct((NCK, 32, CAP), jnp.int32),
              jax.ShapeDtypeStruct((32, L), jnp.int32)],
    mesh=_mesh,
    compiler_params=_params,
    scratch_types=[pltpu.VMEM((ET,), jnp.int32),
                   pltpu.VMEM((ET,), jnp.int32),
                   pltpu.VMEM((CAP,), jnp.int32),
                   pltpu.VMEM((L,), jnp.int32)],
)


def _unpack2(p):
    h0 = plsc.bitcast(p & jnp.int32(-65536), jnp.float32)
    h1 = plsc.bitcast(p << 16, jnp.float32)
    return h0, h1


def _seg_nblk(cnt_v, io16, chunk):
    cv = cnt_v[...]
    return jnp.sum(jnp.where(io16 == chunk, cv, 0)) // 128


def _ex_body(asp, dap, pk, cnts, exv, oud,
             asp_v, dacp_v, dn0, dn1, pkseg, exb2, cnt_v):
    cr = lax.axis_index("c")
    sub = lax.axis_index("s")
    io16 = lax.iota(jnp.int32, L)
    fzero = jnp.zeros((L,), jnp.float32)
    sh16 = jnp.full((L,), 16, jnp.int32)

    pltpu.sync_copy(asp, asp_v)

    for ci in range(NCK // NC):
        chunk = cr * (NCK // NC) + ci
        base = chunk * CH

        def zdn(i, _):
            dn0[pl.ds(L * i, L)] = fzero
            dn1[pl.ds(L * i, L)] = fzero
            return 0
        lax.fori_loop(0, ACC_R // L, zdn, 0)
        pltpu.sync_copy(dap.at[pl.ds(base, CH)], dacp_v.at[pl.ds(0, CH)])

        for mi in range(2):
            m = sub * 2 + mi
            pltpu.sync_copy(cnts.at[m], cnt_v)
            nprs = _seg_nblk(cnt_v, io16, chunk) // 2
            pltpu.sync_copy(pk.at[chunk, m], pkseg)

            def pair(q, _):
                for blk in range(2):
                    for k in range(8):
                        pv = pkseg[pl.ds(q * 256 + blk * 128 + L * k, L)]
                        sv = lax.shift_right_logical(pv, sh16)
                        ldv = pv & LDM
                        pa = plsc.load_gather(asp_v, [sv])
                        pd = plsc.load_gather(dacp_v, [ldv])
                        as0, as1 = _unpack2(pa)
                        ad0, ad1 = _unpack2(pd)
                        a0 = as0 + ad0
                        a1 = as1 + ad1
                        a0 = jnp.where(a0 > 0, a0, 0.2 * a0)
                        a1 = jnp.where(a1 > 0, a1, 0.2 * a1)
                        e0 = jnp.exp(a0)
                        e1 = jnp.exp(a1)
                        exb2[blk, 0, pl.ds(L * k, L)] = e0
                        exb2[blk, 1, pl.ds(L * k, L)] = e1
                        plsc.addupdate_scatter(dn0, [ldv], e0)
                        plsc.addupdate_scatter(dn1, [ldv], e1)
                pltpu.sync_copy(exb2, exv.at[chunk, m, pl.ds(2 * q, 2)])
                return 0
            lax.fori_loop(0, nprs, pair, 0)

        pltpu.sync_copy(dn0, oud.at[sub, 0, chunk])
        pltpu.sync_copy(dn1, oud.at[sub, 1, chunk])


def _msg_body(hx, exv, pk, cnts, om,
              g0, g1, src_u, ld2, pk2, exb2, cnt_v, acc,
              sem0, sem1, scat0, scat1):
    cr = lax.axis_index("c")
    sub = lax.axis_index("s")
    io16 = lax.iota(jnp.int32, L)
    fzero = jnp.zeros((L,), jnp.float32)
    sh16 = jnp.full((L,), 16, jnp.int32)
    zi = jnp.zeros((L,), jnp.int32)
    oi = jnp.ones((L,), jnp.int32)

    def srow(g, exref):
        def body(e, _):
            ev = zi + e
            x0 = plsc.load_gather(exref, [zi, ev])
            x1 = plsc.load_gather(exref, [oi, ev])
            for k in range(4):
                g[e, pl.ds(L * k, L)] = g[e, pl.ds(L * k, L)] * x0
            for k in range(4, 8):
                g[e, pl.ds(L * k, L)] = g[e, pl.ds(L * k, L)] * x1
            return 0
        lax.fori_loop(0, 128, body, 0)

    for ci in range(NCK // NC):
        chunk = cr * (NCK // NC) + ci
        r0 = sub * TPR
        def zg(i, _):
            for k in range(8):
                g0[i, pl.ds(L * k, L)] = fzero
            return 0
        lax.fori_loop(0, 128, zg, 0)
        for k in range(3):
            pltpu.sync_copy(g0, acc.at[pl.ds(r0 + 128 * k, 128)])
        pltpu.sync_copy(g0.at[pl.ds(0, 16)], acc.at[pl.ds(r0 + 384, 16)])
        plsc.subcore_barrier()

        for mi in range(2):
            m = sub * 2 + mi
            pltpu.sync_copy(cnts.at[m], cnt_v)
            nprs = _seg_nblk(cnt_v, io16, chunk) // 2

            def pair(q, _):
                pltpu.sync_copy(pk.at[chunk, m, pl.ds(256 * q, 256)], pk2)
                pltpu.sync_copy(exv.at[chunk, m, pl.ds(2 * q, 2)], exb2)
                for k in range(16):
                    pv = pk2[pl.ds(L * k, L)]
                    src_u[pl.ds(L * k, L)] = lax.shift_right_logical(pv, sh16)
                    ld2[k // 8, pl.ds(L * (k % 8), L)] = pv & LDM
                cpa = pltpu.async_copy(hx.at[src_u.at[pl.ds(0, 128)]], g0,
                                       sem0)
                cpb = pltpu.async_copy(hx.at[src_u.at[pl.ds(128, 128)]], g1,
                                       sem1)
                cpa.wait()
                srow(g0, exb2.at[0])
                sca = pltpu.async_copy(g0, acc.at[ld2.at[0]], scat0, add=True)
                cpb.wait()
                srow(g1, exb2.at[1])
                scb = pltpu.async_copy(g1, acc.at[ld2.at[1]], scat1, add=True)
                sca.wait()
                scb.wait()
                return 0
            lax.fori_loop(0, nprs, pair, 0)

        plsc.subcore_barrier()
        for k in range(3):
            pltpu.sync_copy(acc.at[pl.ds(r0 + 128 * k, 128)],
                            om.at[chunk, pl.ds(r0 + 128 * k, 128)])
        pltpu.sync_copy(acc.at[pl.ds(r0 + 384, 16)],
                        om.at[chunk, pl.ds(r0 + 384, 16)])


_ex_kernel = pl.kernel(
    _ex_body,
    out_type=[jax.ShapeDtypeStruct((NCK, 32, NBLK, 2, 128), jnp.float32),
              jax.ShapeDtypeStruct((NS, 2, NCK, ACC_R), jnp.float32)],
    mesh=_mesh,
    compiler_params=_params,
    scratch_types=[pltpu.VMEM((NP,), jnp.int32),
                   pltpu.VMEM((CH + 8,), jnp.int32),
                   pltpu.VMEM((ACC_R,), jnp.float32),
                   pltpu.VMEM((ACC_R,), jnp.float32),
                   pltpu.VMEM((CAP,), jnp.int32),
                   pltpu.VMEM((2, 2, 128), jnp.float32),
                   pltpu.VMEM((L,), jnp.int32)],
)

_msg_kernel = pl.kernel(
    _msg_body,
    out_type=[jax.ShapeDtypeStruct((NCK, ACC_R, HID), jnp.float32)],
    mesh=_mesh,
    compiler_params=_params,
    scratch_types=[pltpu.VMEM((128, HID), jnp.float32),
                   pltpu.VMEM((128, HID), jnp.float32),
                   pltpu.VMEM((256,), jnp.int32),
                   pltpu.VMEM((2, 128), jnp.int32),
                   pltpu.VMEM((256,), jnp.int32),
                   pltpu.VMEM((2, 2, 128), jnp.float32),
                   pltpu.VMEM((L,), jnp.int32),
                   pltpu.VMEM_SHARED((ACC_R, HID), jnp.float32),
                   pltpu.SemaphoreType.DMA,
                   pltpu.SemaphoreType.DMA,
                   pltpu.SemaphoreType.DMA,
                   pltpu.SemaphoreType.DMA],
)


def _pack2(a0, a1):
    b0 = lax.bitcast_convert_type(a0.astype(jnp.bfloat16),
                                  jnp.uint16).astype(jnp.uint32)
    b1 = lax.bitcast_convert_type(a1.astype(jnp.bfloat16),
                                  jnp.uint16).astype(jnp.uint32)
    return lax.bitcast_convert_type((b0 << 16) | b1, jnp.int32)


def _tables_from_h(h, asp, asim, adp, adim):
    h3 = h.reshape(h.shape[0], 2, DH)
    a_sp = (h3 * asp).sum(-1)
    a_sim = (h3 * asim).sum(-1)
    d_sp = (h3 * adp).sum(-1)
    d_sim = (h3 * adim).sum(-1)
    return (h, _pack2(a_sp[:, 0], a_sp[:, 1]),
            _pack2(a_sim[:, 0], a_sim[:, 1]),
            _pack2(d_sp[:, 0], d_sp[:, 1]),
            _pack2(d_sim[:, 0], d_sim[:, 1]))


def _prep_body(x_ref, pw_ref, pb_ref, asp_ref, asim_ref, adp_ref, adim_ref,
               hx_ref, s1_ref, s2_ref, d1_ref, d2_ref):
    h = jnp.dot(x_ref[...], pw_ref[...],
                preferred_element_type=jnp.float32) + pb_ref[...]
    hx, s1, s2, d1, d2 = _tables_from_h(h, asp_ref[...], asim_ref[...],
                                        adp_ref[...], adim_ref[...])
    hx_ref[...] = hx
    s1_ref[...] = s1
    s2_ref[...] = s2
    d1_ref[...] = d1
    d2_ref[...] = d2


_att_spec = pl.BlockSpec((2, DH), lambda i: (0, 0))
_vec_spec = pl.BlockSpec((NB,), lambda i: (i,))
_mat_spec = pl.BlockSpec((NB, HID), lambda i: (i, 0))
_w_spec = pl.BlockSpec((HID, HID), lambda i: (0, 0))
_b_spec = pl.BlockSpec((HID,), lambda i: (0,))
_tab_out = ([_mat_spec] + [_vec_spec] * 4,
            [jax.ShapeDtypeStruct((NP, HID), jnp.float32)]
            + [jax.ShapeDtypeStruct((NP,), jnp.int32)] * 4)


def _prep(x, pw, pb, asp, asim, adp, adim):
    return pl.pallas_call(
        _prep_body,
        grid=(GB,),
        in_specs=[_mat_spec, _w_spec, _b_spec,
                  _att_spec, _att_spec, _att_spec, _att_spec],
        out_specs=_tab_out[0],
        out_shape=_tab_out[1],
    )(x, pw, pb, asp, asim, adp, adim)


def _normalize(om, dn):
    dsum = dn.sum(0)
    d0 = dsum[0][:, None]
    d1 = dsum[1][:, None]
    denom = jnp.concatenate([jnp.broadcast_to(d0, (NB, DH)),
                             jnp.broadcast_to(d1, (NB, DH))], axis=1)
    return jax.nn.relu(om / jnp.clip(denom, 1e-16))


def _combw_body(om_sp_ref, dn_sp_ref, om_sim_ref, dn_sim_ref, kw_ref, kb_ref,
                o_sp_ref, o_sim_ref, w_ref, wacc):
    i = pl.program_id(0)
    o_sp = _normalize(om_sp_ref[...], dn_sp_ref[...])
    o_sim = _normalize(om_sim_ref[...], dn_sim_ref[...])
    o_sp_ref[...] = o_sp
    o_sim_ref[...] = o_sim
    rows = i * NB + lax.broadcasted_iota(jnp.int32, (NB, 1), 0)
    mask = (rows < NN).astype(jnp.float32)
    t_sp = jnp.tanh(jnp.dot(o_sp, kw_ref[...],
                            preferred_element_type=jnp.float32) + kb_ref[...])
    t_sim = jnp.tanh(jnp.dot(o_sim, kw_ref[...],
                             preferred_element_type=jnp.float32) + kb_ref[...])
    p_sp = (t_sp * mask).sum(0, keepdims=True)
    p_sim = (t_sim * mask).sum(0, keepdims=True)

    @pl.when(i == 0)
    def _():
        wacc[...] = jnp.zeros_like(wacc)
    wacc[...] += jnp.concatenate([p_sp, p_sim], axis=0)

    @pl.when(i == GB - 1)
    def _():
        w_ref[...] = wacc[...]


_dn_spec = pl.BlockSpec((NS, 2, NB), lambda i: (0, 0, i))


def _combw(om_sp, dn_sp, om_sim, dn_sim, kw, kb):
    return pl.pallas_call(
        _combw_body,
        grid=(GB,),
        in_specs=[_mat_spec, _dn_spec, _mat_spec, _dn_spec, _w_spec, _b_spec],
        out_specs=[_mat_spec, _mat_spec,
                   pl.BlockSpec((2, HID), lambda i: (0, 0))],
        out_shape=[jax.ShapeDtypeStruct((NP, HID), jnp.float32),
                   jax.ShapeDtypeStruct((NP, HID), jnp.float32),
                   jax.ShapeDtypeStruct((2, HID), jnp.float32)],
        scratch_shapes=[pltpu.VMEM((2, HID), jnp.float32)],
    )(om_sp, dn_sp, om_sim, dn_sim, kw, kb)


def _sem_attn(wsum, q):
    wm = wsum / NN
    s = (wm * q).sum(-1)
    s = s - jnp.max(s)
    e = jnp.exp(s)
    a = e / e.sum()
    return a[0], a[1]


def _fuse1_body(osp_ref, osim_ref, ws_ref, q_ref, pw_ref, pb_ref, asp_ref,
                asim_ref, adp_ref, adim_ref, hx_ref, s1_ref, s2_ref, d1_ref,
                d2_ref):
    a0, a1 = _sem_attn(ws_ref[...], q_ref[...])
    h1 = jax.nn.relu(a0 * osp_ref[...] + a1 * osim_ref[...])
    h = jnp.dot(h1, pw_ref[...],
                preferred_element_type=jnp.float32) + pb_ref[...]
    hx, s1, s2, d1, d2 = _tables_from_h(h, asp_ref[...], asim_ref[...],
                                        adp_ref[...], adim_ref[...])
    hx_ref[...] = hx
    s1_ref[...] = s1
    s2_ref[...] = s2
    d1_ref[...] = d1
    d2_ref[...] = d2


_ws_spec = pl.BlockSpec((2, HID), lambda i: (0, 0))


def _fuse1(osp, osim, wsum, q, pw, pb, asp, asim, adp, adim):
    return pl.pallas_call(
        _fuse1_body,
        grid=(GB,),
        in_specs=[_mat_spec, _mat_spec, _ws_spec, _b_spec, _w_spec, _b_spec,
                  _att_spec, _att_spec, _att_spec, _att_spec],
        out_specs=_tab_out[0],
        out_shape=_tab_out[1],
    )(osp, osim, wsum, q, pw, pb, asp, asim, adp, adim)


def _fuse2_body(osp_ref, osim_ref, ws_ref, q_ref, lw_ref, lb_ref, o_ref):
    a0, a1 = _sem_attn(ws_ref[...], q_ref[...])
    h2 = jax.nn.relu(a0 * osp_ref[...] + a1 * osim_ref[...])
    o_ref[...] = jnp.dot(h2, lw_ref[...],
                         preferred_element_type=jnp.float32) + lb_ref[...]


def _fuse2(osp, osim, wsum, q, lw, lb):
    return pl.pallas_call(
        _fuse2_body,
        grid=(GB,),
        in_specs=[_mat_spec, _mat_spec, _ws_spec, _b_spec,
                  pl.BlockSpec((HID, OUTD), lambda i: (0, 0)),
                  pl.BlockSpec((OUTD,), lambda i: (0,))],
        out_specs=pl.BlockSpec((NB, OUTD), lambda i: (i, 0)),
        out_shape=jax.ShapeDtypeStruct((NP, OUTD), jnp.float32),
    )(osp, osim, wsum, q, lw, lb)


def _prep_edges(ei):
    src = ei[0].reshape(32, EE // 32)
    dst = ei[1].reshape(32, EE // 32)
    pad_d = jnp.full((32, ET - EE // 32), 1 << 30, jnp.int32)
    pad_s = jnp.zeros((32, ET - EE // 32), jnp.int32)
    return (jnp.concatenate([src, pad_s], axis=1),
            jnp.concatenate([dst, pad_d], axis=1))


def _om_flat(om):
    return om[:, :CH].reshape(NP, HID)


def _dn_flat(oud):
    return oud[:, :, :, :CH].reshape(NS, 2, NP)


def kernel(x_cell, edge_index_spatial, edge_index_similar, proj_w1, proj_b1,
           att_src_sp1, att_dst_sp1, att_src_sim1, att_dst_sim1, k_w1, k_b1,
           q1, proj_w2, proj_b2, att_src_sp2, att_dst_sp2, att_src_sim2,
           att_dst_sim2, k_w2, k_b2, q2, lin_w, lin_b):
    xp = jnp.pad(x_cell, ((0, NP - NN), (0, 0)))
    bins = []
    for ei in (edge_index_spatial, edge_index_similar):
        s2, d2 = _prep_edges(ei)
        pk, cn = _bin_kernel(s2, d2)
        bins.append((pk, cn))

    atts1 = [a.reshape(2, DH) for a in
             (att_src_sp1, att_src_sim1, att_dst_sp1, att_dst_sim1)]
    atts2 = [a.reshape(2, DH) for a in
             (att_src_sp2, att_src_sim2, att_dst_sp2, att_dst_sim2)]

    def layer(hx, s1, s2, d1, d2, kw, kb):
        asps = (s1, s2)
        daps = (d1, d2)
        res = []
        for r in range(2):
            exv, oud = _ex_kernel(asps[r], daps[r], *bins[r])
            om, = _msg_kernel(hx, exv, *bins[r])
            res.append((om, oud))
        return _combw(_om_flat(res[0][0]), _dn_flat(res[0][1]),
                      _om_flat(res[1][0]), _dn_flat(res[1][1]), kw, kb)

    tabs1 = _prep(xp, proj_w1, proj_b1, *atts1)
    osp1, osim1, w1 = layer(*tabs1, k_w1, k_b1)
    tabs2 = _fuse1(osp1, osim1, w1, q1, proj_w2, proj_b2, *atts2)
    osp2, osim2, w2 = layer(*tabs2, k_w2, k_b2)
    out = _fuse2(osp2, osim2, w2, q2, lin_w, lin_b)
    return out[:NN]
